# baseline (speedup 1.0000x reference)
import dataclasses

import jax
import jax.numpy as jnp
from jax import lax
from jax.experimental import pallas as pl
from jax.experimental import pallas as pl
from jax.experimental.pallas import tpu as pltpu
from jax.experimental.pallas import tpu_sc as plsc

VOCAB_ROWS = 1000000
BATCH = 16384
FIELDS = 26
N_EMBED = 64
NUM_IDX = BATCH * FIELDS

NUM_CORES = 2
NUM_SUBCORES = 16
NUM_WORKERS = NUM_CORES * NUM_SUBCORES

K_BATCH = 8
W_ROWS = K_BATCH * FIELDS
WINDOWS_PER_WORKER = BATCH // (K_BATCH * NUM_WORKERS)

LANES = 16


def _sc_gather(scr, idxh_windows, off_windows):
    mesh = plsc.VectorSubcoreMesh(core_axis_name="core", subcore_axis_name="subcore")
    cp = pltpu.CompilerParams()
    if "needs_layout_passes" in pltpu.CompilerParams.__dataclass_fields__:
        cp = dataclasses.replace(cp, needs_layout_passes=False)

    @pl.kernel(
        out_type=jax.ShapeDtypeStruct((BATCH, FIELDS, N_EMBED), jnp.float32),
        mesh=mesh,
        compiler_params=cp,
        scratch_types=[
            pltpu.VMEM((W_ROWS,), jnp.int32),
            pltpu.VMEM((W_ROWS,), jnp.int32),
            pltpu.VMEM((W_ROWS,), jnp.int32),
            pltpu.VMEM((W_ROWS,), jnp.int32),
            pltpu.VMEM((W_ROWS, 2 * N_EMBED), jnp.float32),
            pltpu.VMEM((W_ROWS, 2 * N_EMBED), jnp.float32),
            pltpu.VMEM((W_ROWS, N_EMBED), jnp.float32),
            pltpu.VMEM((W_ROWS, N_EMBED), jnp.float32),
            pltpu.SemaphoreType.DMA,
            pltpu.SemaphoreType.DMA,
            pltpu.SemaphoreType.DMA,
            pltpu.SemaphoreType.DMA,
        ],
    )
    def kern(scr_hbm, idxh_hbm, off_hbm, out_hbm,
             idxh_v0, idxh_v1, off_v0, off_v1, g_v0, g_v1, o_v0, o_v1,
             gsem0, gsem1, osem0, osem1):
        wid = lax.axis_index("core") * NUM_SUBCORES + lax.axis_index("subcore")
        w_base = wid * WINDOWS_PER_WORKER

        bufs = ((idxh_v0, off_v0, g_v0, o_v0, gsem0, osem0),
                (idxh_v1, off_v1, g_v1, o_v1, gsem1, osem1))

        def start_gather(w, idxh_v, off_v, g_v, gsem):
            pltpu.sync_copy(idxh_hbm.at[w_base + w], idxh_v)
            pltpu.sync_copy(off_hbm.at[w_base + w], off_v)
            pltpu.async_copy(scr_hbm.at[idxh_v], g_v, gsem)

        for b in range(2):
            idxh_v, off_v, g_v, _, gsem, _ = bufs[b]
            start_gather(b, idxh_v, off_v, g_v, gsem)

        def do_window(w, idxh_v, off_v, g_v, o_v, gsem, osem):
            b0 = (w_base + w) * K_BATCH
            pltpu.make_async_copy(scr_hbm.at[idxh_v], g_v, gsem).wait()
            @pl.when(w >= 2)
            def _():
                for j in range(K_BATCH):
                    pltpu.make_async_copy(
                        o_v.at[pl.ds(j * FIELDS, FIELDS)], out_hbm.at[0], osem
                    ).wait()

            iota = lax.iota(jnp.int32, LANES)

            @pl.loop(0, W_ROWS)
            def _(r):
                r16 = jnp.zeros((LANES,), jnp.int32) + r
                off16 = plsc.load_gather(off_v, [r16])
                for k in range(0, N_EMBED, LANES):
                    col = off16 + (iota + k)
                    val = plsc.load_gather(g_v, [r16, col])
                    o_v[r, pl.ds(k, LANES)] = val

            @pl.when(w + 2 < WINDOWS_PER_WORKER)
            def _():
                start_gather(w + 2, idxh_v, off_v, g_v, gsem)

            for j in range(K_BATCH):
                pltpu.async_copy(
                    o_v.at[pl.ds(j * FIELDS, FIELDS)], out_hbm.at[b0 + j], osem
                )

        @pl.loop(0, WINDOWS_PER_WORKER, step=2)
        def _(w):
            for b in range(2):
                idxh_v, off_v, g_v, o_v, gsem, osem = bufs[b]
                do_window(w + b, idxh_v, off_v, g_v, o_v, gsem, osem)

        for b in range(2):
            o_v, osem = bufs[b][3], bufs[b][5]
            for j in range(K_BATCH):
                pltpu.make_async_copy(
                    o_v.at[pl.ds(j * FIELDS, FIELDS)], out_hbm.at[0], osem
                ).wait()

    return kern(scr, idxh_windows, off_windows)


def kernel(discrete, embeddings):
    flat = discrete.astype(jnp.int32).reshape(NUM_IDX // W_ROWS, W_ROWS)
    idxh_windows = flat >> 1
    off_windows = (flat & 1) * N_EMBED
    scr = embeddings.reshape(VOCAB_ROWS // 2, 2 * N_EMBED)
    return _sc_gather(scr, idxh_windows, off_windows)
